# baseline (speedup 1.0000x reference)
import functools

import jax
import jax.numpy as jnp
from jax import lax
from jax.experimental import pallas as pl
from jax.experimental.pallas import tpu as pltpu
from jax.experimental.pallas import tpu_sc as plsc

_NC = 2
_NS = 16
_NW = _NC * _NS
_C = 128


def _mesh():
    return plsc.VectorSubcoreMesh(core_axis_name="c", subcore_axis_name="s")


def _make_deg_fn(n_acc, k):
    stripe = n_acc // _NS

    @functools.partial(
        pl.kernel,
        out_type=jax.ShapeDtypeStruct((_NC * n_acc,), jnp.float32),
        mesh=_mesh(),
        scratch_types=[
            pltpu.VMEM((k, _C), jnp.int32),
            pltpu.VMEM((_C,), jnp.float32),
            pltpu.VMEM_SHARED((n_acc,), jnp.float32),
        ],
    )
    def deg_fn(dst_hbm, zeros_hbm, out_hbm, didx, ones_v, shared):
        cid = lax.axis_index("c")
        sid = lax.axis_index("s")
        w = cid * _NS + sid
        r0 = pl.multiple_of(sid * stripe, stripe)
        pltpu.sync_copy(dst_hbm.at[w], didx)
        for i in range(_C // 16):
            ones_v[pl.ds(i * 16, 16)] = jnp.ones((16,), jnp.float32)
        pltpu.sync_copy(zeros_hbm.at[pl.ds(r0, stripe)],
                        shared.at[pl.ds(r0, stripe)])
        plsc.subcore_barrier()

        def body(j, carry):
            pltpu.sync_copy(ones_v, shared.at[didx.at[j]], add=True)
            return carry

        lax.fori_loop(0, k, body, 0)
        plsc.subcore_barrier()
        ofs = pl.multiple_of(cid * n_acc + r0, stripe)
        pltpu.sync_copy(shared.at[pl.ds(r0, stripe)],
                        out_hbm.at[pl.ds(ofs, stripe)])

    return deg_fn


def _make_scatter_fn(n_acc, k, hid):
    stripe = n_acc // _NS

    @functools.partial(
        pl.kernel,
        out_type=jax.ShapeDtypeStruct((_NC, n_acc, hid), jnp.float32),
        mesh=_mesh(),
        scratch_types=[
            pltpu.VMEM((k, _C), jnp.int32),
            pltpu.VMEM((k, _C), jnp.int32),
            pltpu.VMEM((_C, hid), jnp.float32),
            pltpu.VMEM_SHARED((n_acc, hid), jnp.float32),
            pltpu.SemaphoreType.DMA,
        ],
        compiler_params=pltpu.CompilerParams(use_tc_tiling_on_sc=False),
    )
    def scatter_fn(src_hbm, dst_hbm, table_hbm, zeros_hbm, out_hbm,
                   sidx, didx, rows, shared, sem):
        cid = lax.axis_index("c")
        sid = lax.axis_index("s")
        w = cid * _NS + sid
        r0 = pl.multiple_of(sid * stripe, stripe)
        pltpu.sync_copy(zeros_hbm.at[pl.ds(r0, stripe)],
                        shared.at[pl.ds(r0, stripe)])
        pltpu.sync_copy(src_hbm.at[w], sidx)
        pltpu.sync_copy(dst_hbm.at[w], didx)
        plsc.subcore_barrier()

        def body(j, carry):
            pltpu.async_copy(table_hbm.at[sidx.at[j]], rows, sem).wait()
            pltpu.sync_copy(rows, shared.at[didx.at[j]], add=True)
            return carry

        lax.fori_loop(0, k, body, 0)
        plsc.subcore_barrier()
        pltpu.sync_copy(shared.at[pl.ds(r0, stripe)],
                        out_hbm.at[cid, pl.ds(r0, stripe)])

    return scatter_fn


def _dinv(c0, c1):
    return lax.rsqrt(c0 + c1 + 1.0)


def _layer1_body(x_ref, w1_ref, c0_ref, c1_ref, g1_ref):
    h = jnp.dot(x_ref[...], w1_ref[...], preferred_element_type=jnp.float32)
    g1_ref[...] = h * _dinv(c0_ref[...], c1_ref[...])


def _mid_body(p0_ref, p1_ref, g1_ref, c0_ref, c1_ref, b1_ref, g2_ref):
    d = _dinv(c0_ref[...], c1_ref[...])
    agg = (p0_ref[...] + p1_ref[...] + g1_ref[...]) * d
    h2 = jnp.maximum(agg + b1_ref[...], 0.0)
    g2_ref[...] = h2 * d


def _out_body(p0_ref, p1_ref, g2_ref, c0_ref, c1_ref, w2_ref, b2_ref,
              out_ref):
    d = _dinv(c0_ref[...], c1_ref[...])
    agg = (p0_ref[...] + p1_ref[...] + g2_ref[...]) * d
    out_ref[...] = (
        jnp.dot(agg, w2_ref[...], preferred_element_type=jnp.float32)
        + b2_ref[...]
    )


def _tc(body, out_shape, *args):
    return pl.pallas_call(body, out_shape=out_shape)(*args)


def kernel(x, edge_index, W1, b1, W2, b2):
    f32 = jnp.float32
    n, _ = x.shape
    hid = W1.shape[1]
    d_out = W2.shape[1]
    e = edge_index.shape[1]

    n_acc = (n // (2 * _C) + 1) * 2 * _C
    k = (e + _NW * _C - 1) // (_NW * _C)
    e_pad = _NW * _C * k
    pad_n = e_pad - e

    src = edge_index[0]
    dst = edge_index[1]
    if pad_n:
        ar = jnp.arange(pad_n, dtype=edge_index.dtype)
        src = jnp.concatenate([src, ar % n])
        dst = jnp.concatenate([dst, n + ar % (n_acc - n)])
    src_g = src.reshape(_NW, k, _C)
    dst_g = dst.reshape(_NW, k, _C)

    zeros1 = jnp.zeros((n_acc,), f32)
    zeros2 = jnp.zeros((n_acc, hid), f32)

    deg_fn = _make_deg_fn(n_acc, k)
    scat_fn = _make_scatter_fn(n_acc, k, hid)

    cnt = deg_fn(dst_g, zeros1).reshape(_NC, n_acc)
    c0 = cnt[0, :n].reshape(n, 1)
    c1 = cnt[1, :n].reshape(n, 1)

    g1 = _tc(_layer1_body, jax.ShapeDtypeStruct((n, hid), f32),
             x, W1, c0, c1)
    p1 = scat_fn(src_g, dst_g, g1, zeros2)
    g2 = _tc(_mid_body, jax.ShapeDtypeStruct((n, hid), f32),
             p1[0, :n], p1[1, :n], g1, c0, c1, b1.reshape(1, hid))
    p2 = scat_fn(src_g, dst_g, g2, zeros2)
    out = _tc(_out_body, jax.ShapeDtypeStruct((n, d_out), f32),
              p2[0, :n], p2[1, :n], g2, c0, c1, W2, b2.reshape(1, d_out))
    return out


# speedup vs baseline: 1.0196x; 1.0196x over previous
import functools

import jax
import jax.numpy as jnp
from jax import lax
from jax.experimental import pallas as pl
from jax.experimental.pallas import tpu as pltpu
from jax.experimental.pallas import tpu_sc as plsc

_NC = 2
_NS = 16
_NW = _NC * _NS
_C = 128


def _mesh():
    return plsc.VectorSubcoreMesh(core_axis_name="c", subcore_axis_name="s")


def _make_deg_fn(n_acc, k):
    stripe = n_acc // _NS

    @functools.partial(
        pl.kernel,
        out_type=jax.ShapeDtypeStruct((_NC * n_acc,), jnp.float32),
        mesh=_mesh(),
        scratch_types=[
            pltpu.VMEM((k, _C), jnp.int32),
            pltpu.VMEM((_C,), jnp.float32),
            pltpu.VMEM_SHARED((n_acc,), jnp.float32),
        ],
    )
    def deg_fn(dst_hbm, zeros_hbm, out_hbm, didx, ones_v, shared):
        cid = lax.axis_index("c")
        sid = lax.axis_index("s")
        w = cid * _NS + sid
        r0 = pl.multiple_of(sid * stripe, stripe)
        pltpu.sync_copy(dst_hbm.at[w], didx)
        for i in range(_C // 16):
            ones_v[pl.ds(i * 16, 16)] = jnp.ones((16,), jnp.float32)
        pltpu.sync_copy(zeros_hbm.at[pl.ds(r0, stripe)],
                        shared.at[pl.ds(r0, stripe)])
        plsc.subcore_barrier()

        def body(j, carry):
            pltpu.sync_copy(ones_v, shared.at[didx.at[j]], add=True)
            return carry

        lax.fori_loop(0, k, body, 0)
        plsc.subcore_barrier()
        ofs = pl.multiple_of(cid * n_acc + r0, stripe)
        pltpu.sync_copy(shared.at[pl.ds(r0, stripe)],
                        out_hbm.at[pl.ds(ofs, stripe)])

    return deg_fn


def _make_scatter_fn(n_acc, k, hid):
    stripe = n_acc // _NS

    @functools.partial(
        pl.kernel,
        out_type=jax.ShapeDtypeStruct((_NC, n_acc, hid), jnp.float32),
        mesh=_mesh(),
        scratch_types=[
            pltpu.VMEM((k, _C), jnp.int32),
            pltpu.VMEM((k, _C), jnp.int32),
            pltpu.VMEM((_C, hid), jnp.float32),
            pltpu.VMEM_SHARED((n_acc, hid), jnp.float32),
            pltpu.SemaphoreType.DMA,
        ],
        compiler_params=pltpu.CompilerParams(use_tc_tiling_on_sc=False),
    )
    def scatter_fn(src_hbm, dst_hbm, table_hbm, zeros_hbm, out_hbm,
                   sidx, didx, rows, shared, sem):
        cid = lax.axis_index("c")
        sid = lax.axis_index("s")
        w = cid * _NS + sid
        r0 = pl.multiple_of(sid * stripe, stripe)
        pltpu.sync_copy(zeros_hbm.at[pl.ds(r0, stripe)],
                        shared.at[pl.ds(r0, stripe)])
        pltpu.sync_copy(src_hbm.at[w], sidx)
        pltpu.sync_copy(dst_hbm.at[w], didx)
        plsc.subcore_barrier()

        def body(j, carry):
            pltpu.async_copy(table_hbm.at[sidx.at[j]], rows, sem).wait()
            pltpu.sync_copy(rows, shared.at[didx.at[j]], add=True)
            return carry

        lax.fori_loop(0, k, body, 0)
        plsc.subcore_barrier()
        pltpu.sync_copy(shared.at[pl.ds(r0, stripe)],
                        out_hbm.at[cid, pl.ds(r0, stripe)])

    return scatter_fn


def _dinv_body(c0_ref, c1_ref, dinv_ref):
    cnt = c0_ref[...] + c1_ref[...]
    dinv_ref[...] = lax.rsqrt(cnt + 1.0)


def _layer1_body(x_ref, w1_ref, dinv_ref, g1_ref):
    h = jnp.dot(x_ref[...], w1_ref[...], preferred_element_type=jnp.float32)
    g1_ref[...] = h * dinv_ref[...]


def _mid_body(p0_ref, p1_ref, g1_ref, dinv_ref, b1_ref, g2_ref):
    agg = (p0_ref[...] + p1_ref[...] + g1_ref[...]) * dinv_ref[...]
    h2 = jnp.maximum(agg + b1_ref[...], 0.0)
    g2_ref[...] = h2 * dinv_ref[...]


def _out_body(p0_ref, p1_ref, g2_ref, dinv_ref, w2_ref, b2_ref, out_ref):
    agg = (p0_ref[...] + p1_ref[...] + g2_ref[...]) * dinv_ref[...]
    out_ref[...] = (
        jnp.dot(agg, w2_ref[...], preferred_element_type=jnp.float32)
        + b2_ref[...]
    )


def _tc(body, out_shape, *args):
    return pl.pallas_call(body, out_shape=out_shape)(*args)


def kernel(x, edge_index, W1, b1, W2, b2):
    f32 = jnp.float32
    n, _ = x.shape
    hid = W1.shape[1]
    d_out = W2.shape[1]
    e = edge_index.shape[1]

    n_acc = (n // (2 * _C) + 1) * 2 * _C
    k = (e + _NW * _C - 1) // (_NW * _C)
    e_pad = _NW * _C * k
    pad_n = e_pad - e

    src = edge_index[0]
    dst = edge_index[1]
    if pad_n:
        ar = jnp.arange(pad_n, dtype=edge_index.dtype)
        src = jnp.concatenate([src, ar % n])
        dst = jnp.concatenate([dst, n + ar % (n_acc - n)])
    src_g = src.reshape(_NW, k, _C)
    dst_g = dst.reshape(_NW, k, _C)

    zeros1 = jnp.zeros((n_acc,), f32)
    zeros2 = jnp.zeros((n_acc, hid), f32)

    deg_fn = _make_deg_fn(n_acc, k)
    scat_fn = _make_scatter_fn(n_acc, k, hid)

    cnt = deg_fn(dst_g, zeros1).reshape(_NC, n_acc)
    dinv = _tc(_dinv_body, jax.ShapeDtypeStruct((1, n), f32),
               cnt[0:1, :n], cnt[1:2, :n])
    dinv_c = dinv.reshape(n, 1)

    g1 = _tc(_layer1_body, jax.ShapeDtypeStruct((n, hid), f32),
             x, W1, dinv_c)
    p1 = scat_fn(src_g, dst_g, g1, zeros2)
    g2 = _tc(_mid_body, jax.ShapeDtypeStruct((n, hid), f32),
             p1[0, :n], p1[1, :n], g1, dinv_c, b1.reshape(1, hid))
    p2 = scat_fn(src_g, dst_g, g2, zeros2)
    out = _tc(_out_body, jax.ShapeDtypeStruct((n, d_out), f32),
              p2[0, :n], p2[1, :n], g2, dinv_c, W2, b2.reshape(1, d_out))
    return out
